# baseline (speedup 1.0000x reference)
import functools

import jax
import jax.numpy as jnp
from jax import lax
from jax.experimental import pallas as pl
from jax.experimental.pallas import tpu as pltpu
from jax.experimental.pallas import tpu_sc as plsc

_B = 16
_G = 978
_GP = 1024
_GVD = 200
_H = 4
_C = 64
_HC = _H * _C
_E = 20000
_GEXFULL = 12328
_ROWS_PER_TILE = _GP // 32


def _sc_graph_prep(ppi, gex_idx_padded, gex_x1):
    mesh = plsc.VectorSubcoreMesh(core_axis_name="c", subcore_axis_name="s")

    @functools.partial(
        pl.kernel,
        mesh=mesh,
        compiler_params=pltpu.CompilerParams(needs_layout_passes=False),
        out_type=[
            jax.ShapeDtypeStruct((_GP * _GP,), jnp.float32),
            jax.ShapeDtypeStruct((_B, _GP), jnp.float32),
        ],
        scratch_types=[
            pltpu.VMEM((_E,), jnp.int32),
            pltpu.VMEM((_E,), jnp.int32),
            pltpu.VMEM((_ROWS_PER_TILE * _GP,), jnp.float32),
            pltpu.VMEM((_GP,), jnp.int32),
            pltpu.VMEM((_GEXFULL,), jnp.float32),
            pltpu.VMEM((_GP,), jnp.float32),
        ],
    )
    def k(ppi_hbm, idx_hbm, gex_hbm, adj_hbm, gexg_hbm, sv, dv, acc, idxv, rowv, outv):
        wid = lax.axis_index("s") * 2 + lax.axis_index("c")
        lo = wid * _ROWS_PER_TILE
        zeros16 = jnp.zeros((16,), jnp.float32)
        ones16 = jnp.ones((16,), jnp.float32)
        iota16 = lax.iota(jnp.int32, 16)

        def zrow(i, carry):
            acc[pl.ds(i * 16, 16)] = zeros16
            return carry

        lax.fori_loop(0, _ROWS_PER_TILE * _GP // 16, zrow, 0)

        pltpu.sync_copy(ppi_hbm.at[0], sv)
        pltpu.sync_copy(ppi_hbm.at[1], dv)

        def edge_body(i, carry):
            ss = sv[pl.ds(i * 16, 16)]
            dd = dv[pl.ds(i * 16, 16)]
            m = (dd >= lo) & (dd < lo + _ROWS_PER_TILE)
            flat = jnp.where(m, (dd - lo) * _GP + ss, 0)
            plsc.addupdate_scatter(acc, [flat], ones16, mask=m)
            return carry

        lax.fori_loop(0, _E // 16, edge_body, 0)

        for j in range(_ROWS_PER_TILE // 16):
            ii = lo + j * 16 + iota16
            m = ii < _G
            flat = jnp.where(m, (ii - lo) * _GP + ii, 0)
            plsc.addupdate_scatter(acc, [flat], ones16, mask=m)

        pltpu.sync_copy(acc, adj_hbm.at[pl.ds(lo * _GP, _ROWS_PER_TILE * _GP)])

        @pl.when(wid < _B)
        def _():
            pltpu.sync_copy(idx_hbm, idxv)
            pltpu.sync_copy(gex_hbm.at[wid], rowv)

            def gather_body(i, carry):
                ii = idxv[pl.ds(i * 16, 16)]
                outv[pl.ds(i * 16, 16)] = plsc.load_gather(rowv, [ii])
                return carry

            lax.fori_loop(0, _GP // 16, gather_body, 0)
            pltpu.sync_copy(outv, gexg_hbm.at[wid])

    return k(ppi, gex_idx_padded, gex_x1)


def _bn(x, g, b):
    mu = jnp.mean(x, axis=0, keepdims=True)
    v = jnp.mean((x - mu) ** 2, axis=0, keepdims=True)
    return (x - mu) * lax.rsqrt(v + 1e-5) * g + b


def _prep_body(drug, gexg, dose, dur, adj, dW1, db1, dW2, db2, dW3, db3,
               g1, b1, g2, b2, g3, b3, g4, b4,
               de_o, gexn_o, dose2_o, dur2_o, logA_o):
    a = adj[...]
    logA_o[...] = jnp.where(a > 0.0, jnp.log(a), -1e30)
    d0 = _bn(drug[...], g1[...], b1[...])
    h = jnp.maximum(jnp.dot(d0, dW1[...], preferred_element_type=jnp.float32) + db1[...], 0.0)
    h = jnp.maximum(jnp.dot(h, dW2[...], preferred_element_type=jnp.float32) + db2[...], 0.0)
    h = jnp.maximum(jnp.dot(h, dW3[...], preferred_element_type=jnp.float32) + db3[...], 0.0)
    de_o[...] = h
    gexn_o[...] = _bn(gexg[...], g2[...], b2[...])
    dose2_o[...] = _bn(dose[...], g3[...], b3[...])
    dur2_o[...] = _bn(dur[...], g4[...], b4[...])


def _gat_layer(x, W, a_s, a_d, gb, logA):
    xp = jnp.dot(x, W, preferred_element_type=jnp.float32)
    outs = []
    for h in range(_H):
        xph = xp[:, h * _C:(h + 1) * _C]
        edh = jnp.sum(xph * a_d[:, h * _C:(h + 1) * _C], axis=1, keepdims=True)
        esr = lax.dot_general(a_s[:, h * _C:(h + 1) * _C], xph,
                              (((1,), (1,)), ((), ())),
                              preferred_element_type=jnp.float32)
        e = esr + edh
        em = jnp.maximum(e, 0.2 * e) + logA
        mh = jnp.max(em, axis=1, keepdims=True)
        ex = jnp.exp(em - mh)
        den = jnp.sum(ex, axis=1, keepdims=True)
        outs.append(jnp.dot(ex, xph, preferred_element_type=jnp.float32)
                    / (den + 1e-16))
    return jnp.maximum(jnp.concatenate(outs, axis=1) + gb, 0.0)


def _fused_body(gexnT, g2vp, logA_ref, gW0, a0s, a0d, gb0, gW1, a1s, a1d, gb1,
                rW1, rb1, rW2, rb2, rW3, rb3,
                de, dose2, dur2, pW1a, pW1b, pW1c, pb1, pW2, pb2, pW3, pb3,
                out_o):
    b = pl.program_id(0)
    logA = logA_ref[...]
    rowid = lax.broadcasted_iota(jnp.int32, (_B, 1), 0)
    oh = (rowid == b).astype(jnp.float32)
    xcol = jnp.dot(gexnT[...], oh, preferred_element_type=jnp.float32)
    ge = xcol * g2vp[...]

    h0 = _gat_layer(ge, gW0[...], a0s[...], a0d[...], gb0[...], logA)
    h1 = _gat_layer(h0, gW1[...], a1s[...], a1d[...], gb1[...], logA)
    hcat = jnp.concatenate([h0, h1], axis=1)

    ro = lax.dot_general(hcat, rW1[...], (((0,), (0,)), ((), ())),
                         preferred_element_type=jnp.float32)
    ro = jnp.maximum(ro + rb1[...], 0.0)
    ro = jnp.maximum(jnp.dot(ro, rW2[...], preferred_element_type=jnp.float32) + rb2[...], 0.0)
    ro = jnp.maximum(jnp.dot(ro, rW3[...], preferred_element_type=jnp.float32) + rb3[...], 0.0)

    t = lax.dot_general(ro, pW1b[...], (((0,), (0,)), ((), ())),
                        preferred_element_type=jnp.float32)
    t = t + jnp.dot(de[0], pW1a[...], preferred_element_type=jnp.float32)
    t = t + dose2[0] * pW1c[0:1, :] + dur2[0] * pW1c[1:2, :]
    t = jnp.maximum(t + pb1[...], 0.0)
    t = jnp.maximum(jnp.dot(t, pW2[...], preferred_element_type=jnp.float32) + pb2[...], 0.0)
    res = jnp.dot(t, pW3[...], preferred_element_type=jnp.float32) + pb3[...]
    out_o[...] = res.reshape(1, 1, res.shape[1])


def kernel(gex_x1, drug_input, dose, duration, get_gex_idxs, ppi_adj, g2v,
           dW1, db1, dW2, db2, dW3, db3, gW0, as0, ad0, gb0, gW1, as1, ad1, gb1,
           rW1, rb1, rW2, rb2, rW3, rb3, pW1, pb1, pW2, pb2, pW3, pb3,
           bn1_g, bn1_b, bn2_g, bn2_b, bn3_g, bn3_b, bn4_g, bn4_b):
    f32 = jnp.float32

    ppi = ppi_adj.astype(jnp.int32)
    idxp = jnp.pad(get_gex_idxs.astype(jnp.int32), (0, _GP - _G))
    adj_flat, gexg = _sc_graph_prep(ppi, idxp, gex_x1.astype(f32))
    adj = adj_flat.reshape(_GP, _GP)

    de, gexn, dose2, dur2, logA = pl.pallas_call(
        _prep_body,
        out_shape=[
            jax.ShapeDtypeStruct((_B, 128), f32),
            jax.ShapeDtypeStruct((_B, _GP), f32),
            jax.ShapeDtypeStruct((_B, 1), f32),
            jax.ShapeDtypeStruct((_B, 1), f32),
            jax.ShapeDtypeStruct((_GP, _GP), f32),
        ],
    )(
        drug_input, gexg, dose.reshape(_B, 1), duration.reshape(_B, 1), adj,
        dW1, db1.reshape(1, -1), dW2, db2.reshape(1, -1), dW3, db3.reshape(1, -1),
        bn1_g.reshape(1, -1), bn1_b.reshape(1, -1),
        jnp.pad(bn2_g, (0, _GP - _G)).reshape(1, _GP),
        jnp.pad(bn2_b, (0, _GP - _G)).reshape(1, _GP),
        bn3_g.reshape(1, 1), bn3_b.reshape(1, 1),
        bn4_g.reshape(1, 1), bn4_b.reshape(1, 1),
    )

    gexnT = gexn.T
    g2vp = jnp.pad(g2v, ((0, _GP - _G), (0, 256 - _GVD)))
    gW0p = jnp.pad(gW0, ((0, 256 - _GVD), (0, 0)))
    rW1p = jnp.pad(rW1, ((0, _GP - _G), (0, 0)))
    pW1a = pW1[0:128]
    pW1b = pW1[128:128 + 512]
    pW1c = pW1[128 + 512:]
    r1 = rW1.shape[1]
    r2 = rW2.shape[1]
    p1 = pW1.shape[1]
    p2 = pW2.shape[1]
    nc = pW3.shape[1]

    const2 = lambda blk: pl.BlockSpec(blk, lambda b: (0, 0))
    perb3 = lambda blk: pl.BlockSpec(blk, lambda b: (b, 0, 0))

    out = pl.pallas_call(
        _fused_body,
        grid=(_B,),
        in_specs=[
            const2((_GP, _B)),
            const2((_GP, 256)),
            const2((_GP, _GP)),
            const2((256, _HC)),
            const2((1, _HC)),
            const2((1, _HC)),
            const2((1, _HC)),
            const2((_HC, _HC)),
            const2((1, _HC)),
            const2((1, _HC)),
            const2((1, _HC)),
            const2((_GP, r1)),
            const2((1, r1)),
            const2((r1, r2)),
            const2((1, r2)),
            const2((r2, 1)),
            const2((1, 1)),
            perb3((1, 1, 128)),
            perb3((1, 1, 1)),
            perb3((1, 1, 1)),
            const2((128, p1)),
            const2((512, p1)),
            const2((2, p1)),
            const2((1, p1)),
            const2((p1, p2)),
            const2((1, p2)),
            const2((p2, nc)),
            const2((1, nc)),
        ],
        out_specs=perb3((1, 1, nc)),
        out_shape=jax.ShapeDtypeStruct((_B, 1, nc), f32),
    )(
        gexnT, g2vp, logA, gW0p, as0.reshape(1, _HC), ad0.reshape(1, _HC),
        gb0.reshape(1, _HC), gW1, as1.reshape(1, _HC), ad1.reshape(1, _HC),
        gb1.reshape(1, _HC), rW1p, rb1.reshape(1, r1), rW2, rb2.reshape(1, r2),
        rW3, rb3.reshape(1, 1), de.reshape(_B, 1, 128), dose2.reshape(_B, 1, 1),
        dur2.reshape(_B, 1, 1),
        pW1a, pW1b, pW1c, pb1.reshape(1, p1), pW2, pb2.reshape(1, p2),
        pW3, pb3.reshape(1, nc),
    )
    return out.reshape(_B, nc)


# speedup vs baseline: 1.1735x; 1.1735x over previous
import functools

import jax
import jax.numpy as jnp
from jax import lax
from jax.experimental import pallas as pl
from jax.experimental.pallas import tpu as pltpu
from jax.experimental.pallas import tpu_sc as plsc

_B = 16
_G = 978
_GP = 1024
_GVD = 200
_H = 4
_C = 64
_HC = _H * _C
_E = 20000
_GEXFULL = 12328
_ROWS_PER_TILE = _GP // 32


def _sc_graph_prep(ppi, gex_idx_padded, gex_x1):
    mesh = plsc.VectorSubcoreMesh(core_axis_name="c", subcore_axis_name="s")

    @functools.partial(
        pl.kernel,
        mesh=mesh,
        compiler_params=pltpu.CompilerParams(needs_layout_passes=False),
        out_type=[
            jax.ShapeDtypeStruct((_GP * _GP,), jnp.float32),
            jax.ShapeDtypeStruct((_B, _GP), jnp.float32),
        ],
        scratch_types=[
            pltpu.VMEM((_E,), jnp.int32),
            pltpu.VMEM((_E,), jnp.int32),
            pltpu.VMEM((_ROWS_PER_TILE * _GP,), jnp.float32),
            pltpu.VMEM((_GP,), jnp.int32),
            pltpu.VMEM((_GEXFULL,), jnp.float32),
            pltpu.VMEM((_GP,), jnp.float32),
        ],
    )
    def k(ppi_hbm, idx_hbm, gex_hbm, adj_hbm, gexg_hbm, sv, dv, acc, idxv, rowv, outv):
        wid = lax.axis_index("s") * 2 + lax.axis_index("c")
        lo = wid * _ROWS_PER_TILE
        zeros16 = jnp.zeros((16,), jnp.float32)
        ones16 = jnp.ones((16,), jnp.float32)
        iota16 = lax.iota(jnp.int32, 16)

        def zrow(i, carry):
            acc[pl.ds(i * 16, 16)] = zeros16
            return carry

        lax.fori_loop(0, _ROWS_PER_TILE * _GP // 16, zrow, 0)

        pltpu.sync_copy(ppi_hbm.at[0], sv)
        pltpu.sync_copy(ppi_hbm.at[1], dv)

        def edge_body(i, carry):
            ss = sv[pl.ds(i * 16, 16)]
            dd = dv[pl.ds(i * 16, 16)]
            m = (dd >= lo) & (dd < lo + _ROWS_PER_TILE)
            flat = jnp.where(m, (dd - lo) * _GP + ss, 0)
            plsc.addupdate_scatter(acc, [flat], ones16, mask=m)
            return carry

        lax.fori_loop(0, _E // 16, edge_body, 0)

        for j in range(_ROWS_PER_TILE // 16):
            ii = lo + j * 16 + iota16
            m = ii < _G
            flat = jnp.where(m, (ii - lo) * _GP + ii, 0)
            plsc.addupdate_scatter(acc, [flat], ones16, mask=m)

        pltpu.sync_copy(acc, adj_hbm.at[pl.ds(lo * _GP, _ROWS_PER_TILE * _GP)])

        @pl.when(wid < _B)
        def _():
            pltpu.sync_copy(idx_hbm, idxv)
            pltpu.sync_copy(gex_hbm.at[wid], rowv)

            def gather_body(i, carry):
                ii = idxv[pl.ds(i * 16, 16)]
                outv[pl.ds(i * 16, 16)] = plsc.load_gather(rowv, [ii])
                return carry

            lax.fori_loop(0, _GP // 16, gather_body, 0)
            pltpu.sync_copy(outv, gexg_hbm.at[wid])

    return k(ppi, gex_idx_padded, gex_x1)


def _bn(x, g, b):
    mu = jnp.mean(x, axis=0, keepdims=True)
    v = jnp.mean((x - mu) ** 2, axis=0, keepdims=True)
    return (x - mu) * lax.rsqrt(v + 1e-5) * g + b


def _prep_body(drug, gexg, dose, dur, adj, dW1, db1, dW2, db2, dW3, db3,
               g1, b1, g2, b2, g3, b3, g4, b4,
               de_o, gexn_o, dose2_o, dur2_o, logA_o):
    a = adj[...]
    logA_o[...] = jnp.where(a > 0.0, jnp.log(a), -1e30)
    d0 = _bn(drug[...], g1[...], b1[...])
    h = jnp.maximum(jnp.dot(d0, dW1[...], preferred_element_type=jnp.float32) + db1[...], 0.0)
    h = jnp.maximum(jnp.dot(h, dW2[...], preferred_element_type=jnp.float32) + db2[...], 0.0)
    h = jnp.maximum(jnp.dot(h, dW3[...], preferred_element_type=jnp.float32) + db3[...], 0.0)
    de_o[...] = h
    gexn_o[...] = _bn(gexg[...], g2[...], b2[...])
    dose2_o[...] = _bn(dose[...], g3[...], b3[...])
    dur2_o[...] = _bn(dur[...], g4[...], b4[...])


def _gat_layer(x, W, a_s, a_d, gb, logA, ones_col):
    xp = jnp.dot(x, W, preferred_element_type=jnp.float32)
    outs = []
    for h in range(_H):
        xph = xp[:, h * _C:(h + 1) * _C]
        edh = jnp.sum(xph * a_d[:, h * _C:(h + 1) * _C], axis=1, keepdims=True)
        esr = lax.dot_general(a_s[:, h * _C:(h + 1) * _C], xph,
                              (((1,), (1,)), ((), ())),
                              preferred_element_type=jnp.float32)
        e = esr + edh
        em = jnp.maximum(e, 0.2 * e) + logA
        mh = jnp.max(em, axis=1, keepdims=True)
        ex = jnp.exp(em - mh)
        agg = jnp.dot(ex, jnp.concatenate([xph, ones_col], axis=1),
                      preferred_element_type=jnp.float32)
        outs.append(agg[:, :_C] / (agg[:, _C:_C + 1] + 1e-16))
    return jnp.maximum(jnp.concatenate(outs, axis=1) + gb, 0.0)


def _fused_body(gexnT, g2vp, logA_ref, gW0, a0s, a0d, gb0, gW1, a1s, a1d, gb1,
                rW1, rb1, rW2, rb2, rW3, rb3,
                de, dose2, dur2, pW1a, pW1b, pW1c, pb1, pW2, pb2, pW3, pb3,
                out_o):
    b = pl.program_id(0)
    logA = logA_ref[...]
    rowid = lax.broadcasted_iota(jnp.int32, (_B, 1), 0)
    oh = (rowid == b).astype(jnp.float32)
    xcol = jnp.dot(gexnT[...], oh, preferred_element_type=jnp.float32)
    ge = xcol * g2vp[...]

    ones_col = jnp.ones((_GP, 1), jnp.float32)
    h0 = _gat_layer(ge, gW0[...], a0s[...], a0d[...], gb0[...], logA, ones_col)
    h1 = _gat_layer(h0, gW1[...], a1s[...], a1d[...], gb1[...], logA, ones_col)
    hcat = jnp.concatenate([h0, h1], axis=1)

    ro = lax.dot_general(hcat, rW1[...], (((0,), (0,)), ((), ())),
                         preferred_element_type=jnp.float32)
    ro = jnp.maximum(ro + rb1[...], 0.0)
    ro = jnp.maximum(jnp.dot(ro, rW2[...], preferred_element_type=jnp.float32) + rb2[...], 0.0)
    ro = jnp.maximum(jnp.dot(ro, rW3[...], preferred_element_type=jnp.float32) + rb3[...], 0.0)

    t = lax.dot_general(ro, pW1b[...], (((0,), (0,)), ((), ())),
                        preferred_element_type=jnp.float32)
    t = t + jnp.dot(de[0], pW1a[...], preferred_element_type=jnp.float32)
    t = t + dose2[0] * pW1c[0:1, :] + dur2[0] * pW1c[1:2, :]
    t = jnp.maximum(t + pb1[...], 0.0)
    t = jnp.maximum(jnp.dot(t, pW2[...], preferred_element_type=jnp.float32) + pb2[...], 0.0)
    res = jnp.dot(t, pW3[...], preferred_element_type=jnp.float32) + pb3[...]
    out_o[...] = res.reshape(1, 1, res.shape[1])


def kernel(gex_x1, drug_input, dose, duration, get_gex_idxs, ppi_adj, g2v,
           dW1, db1, dW2, db2, dW3, db3, gW0, as0, ad0, gb0, gW1, as1, ad1, gb1,
           rW1, rb1, rW2, rb2, rW3, rb3, pW1, pb1, pW2, pb2, pW3, pb3,
           bn1_g, bn1_b, bn2_g, bn2_b, bn3_g, bn3_b, bn4_g, bn4_b):
    f32 = jnp.float32

    ppi = ppi_adj.astype(jnp.int32)
    idxp = jnp.pad(get_gex_idxs.astype(jnp.int32), (0, _GP - _G))
    adj_flat, gexg = _sc_graph_prep(ppi, idxp, gex_x1.astype(f32))
    adj = adj_flat.reshape(_GP, _GP)

    de, gexn, dose2, dur2, logA = pl.pallas_call(
        _prep_body,
        out_shape=[
            jax.ShapeDtypeStruct((_B, 128), f32),
            jax.ShapeDtypeStruct((_B, _GP), f32),
            jax.ShapeDtypeStruct((_B, 1), f32),
            jax.ShapeDtypeStruct((_B, 1), f32),
            jax.ShapeDtypeStruct((_GP, _GP), f32),
        ],
    )(
        drug_input, gexg, dose.reshape(_B, 1), duration.reshape(_B, 1), adj,
        dW1, db1.reshape(1, -1), dW2, db2.reshape(1, -1), dW3, db3.reshape(1, -1),
        bn1_g.reshape(1, -1), bn1_b.reshape(1, -1),
        jnp.pad(bn2_g, (0, _GP - _G)).reshape(1, _GP),
        jnp.pad(bn2_b, (0, _GP - _G)).reshape(1, _GP),
        bn3_g.reshape(1, 1), bn3_b.reshape(1, 1),
        bn4_g.reshape(1, 1), bn4_b.reshape(1, 1),
    )

    gexnT = gexn.T
    g2vp = jnp.pad(g2v, ((0, _GP - _G), (0, 256 - _GVD)))
    gW0p = jnp.pad(gW0, ((0, 256 - _GVD), (0, 0)))
    rW1p = jnp.pad(rW1, ((0, _GP - _G), (0, 0)))
    pW1a = pW1[0:128]
    pW1b = pW1[128:128 + 512]
    pW1c = pW1[128 + 512:]
    r1 = rW1.shape[1]
    r2 = rW2.shape[1]
    p1 = pW1.shape[1]
    p2 = pW2.shape[1]
    nc = pW3.shape[1]

    const2 = lambda blk: pl.BlockSpec(blk, lambda b: (0, 0))
    perb3 = lambda blk: pl.BlockSpec(blk, lambda b: (b, 0, 0))

    out = pl.pallas_call(
        _fused_body,
        grid=(_B,),
        in_specs=[
            const2((_GP, _B)),
            const2((_GP, 256)),
            const2((_GP, _GP)),
            const2((256, _HC)),
            const2((1, _HC)),
            const2((1, _HC)),
            const2((1, _HC)),
            const2((_HC, _HC)),
            const2((1, _HC)),
            const2((1, _HC)),
            const2((1, _HC)),
            const2((_GP, r1)),
            const2((1, r1)),
            const2((r1, r2)),
            const2((1, r2)),
            const2((r2, 1)),
            const2((1, 1)),
            perb3((1, 1, 128)),
            perb3((1, 1, 1)),
            perb3((1, 1, 1)),
            const2((128, p1)),
            const2((512, p1)),
            const2((2, p1)),
            const2((1, p1)),
            const2((p1, p2)),
            const2((1, p2)),
            const2((p2, nc)),
            const2((1, nc)),
        ],
        out_specs=perb3((1, 1, nc)),
        out_shape=jax.ShapeDtypeStruct((_B, 1, nc), f32),
    )(
        gexnT, g2vp, logA, gW0p, as0.reshape(1, _HC), ad0.reshape(1, _HC),
        gb0.reshape(1, _HC), gW1, as1.reshape(1, _HC), ad1.reshape(1, _HC),
        gb1.reshape(1, _HC), rW1p, rb1.reshape(1, r1), rW2, rb2.reshape(1, r2),
        rW3, rb3.reshape(1, 1), de.reshape(_B, 1, 128), dose2.reshape(_B, 1, 1),
        dur2.reshape(_B, 1, 1),
        pW1a, pW1b, pW1c, pb1.reshape(1, p1), pW2, pb2.reshape(1, p2),
        pW3, pb3.reshape(1, nc),
    )
    return out.reshape(_B, nc)


# speedup vs baseline: 1.2140x; 1.0345x over previous
import functools

import jax
import jax.numpy as jnp
from jax import lax
from jax.experimental import pallas as pl
from jax.experimental.pallas import tpu as pltpu
from jax.experimental.pallas import tpu_sc as plsc

_B = 16
_G = 978
_GP = 1024
_GVD = 200
_H = 4
_C = 64
_HC = _H * _C
_E = 20000
_GEXFULL = 12328
_ROWS_PER_TILE = _GP // 32


def _sc_graph_prep(ppi, gex_idx_padded, gex_x1):
    mesh = plsc.VectorSubcoreMesh(core_axis_name="c", subcore_axis_name="s")

    @functools.partial(
        pl.kernel,
        mesh=mesh,
        compiler_params=pltpu.CompilerParams(needs_layout_passes=False),
        out_type=[
            jax.ShapeDtypeStruct((_GP * _GP,), jnp.float32),
            jax.ShapeDtypeStruct((_B, _GP), jnp.float32),
        ],
        scratch_types=[
            pltpu.VMEM((_E,), jnp.int32),
            pltpu.VMEM((_E,), jnp.int32),
            pltpu.VMEM((_ROWS_PER_TILE * _GP,), jnp.float32),
            pltpu.VMEM((_GP,), jnp.int32),
            pltpu.VMEM((_GEXFULL,), jnp.float32),
            pltpu.VMEM((_GP,), jnp.float32),
        ],
    )
    def k(ppi_hbm, idx_hbm, gex_hbm, adj_hbm, gexg_hbm, sv, dv, acc, idxv, rowv, outv):
        wid = lax.axis_index("s") * 2 + lax.axis_index("c")
        lo = wid * _ROWS_PER_TILE
        zeros16 = jnp.zeros((16,), jnp.float32)
        ones16 = jnp.ones((16,), jnp.float32)
        iota16 = lax.iota(jnp.int32, 16)

        def zrow(i, carry):
            for u in range(4):
                acc[pl.ds((i * 4 + u) * 16, 16)] = zeros16
            return carry

        lax.fori_loop(0, _ROWS_PER_TILE * _GP // 64, zrow, 0)

        pltpu.sync_copy(ppi_hbm.at[0], sv)
        pltpu.sync_copy(ppi_hbm.at[1], dv)

        def edge_chunk(j):
            ss = sv[pl.ds(j * 16, 16)]
            dd = dv[pl.ds(j * 16, 16)]
            m = (dd >= lo) & (dd < lo + _ROWS_PER_TILE)
            flat = jnp.where(m, (dd - lo) * _GP + ss, 0)
            plsc.addupdate_scatter(acc, [flat], ones16, mask=m)

        def edge_body(i, carry):
            for u in range(4):
                edge_chunk(i * 4 + u)
            return carry

        n_chunks = _E // 16
        lax.fori_loop(0, n_chunks // 4, edge_body, 0)
        for j in range((n_chunks // 4) * 4, n_chunks):
            edge_chunk(j)

        for j in range(_ROWS_PER_TILE // 16):
            ii = lo + j * 16 + iota16
            m = ii < _G
            flat = jnp.where(m, (ii - lo) * _GP + ii, 0)
            plsc.addupdate_scatter(acc, [flat], ones16, mask=m)

        pltpu.sync_copy(acc, adj_hbm.at[pl.ds(lo * _GP, _ROWS_PER_TILE * _GP)])

        @pl.when(wid < _B)
        def _():
            pltpu.sync_copy(idx_hbm, idxv)
            pltpu.sync_copy(gex_hbm.at[wid], rowv)

            def gather_body(i, carry):
                ii = idxv[pl.ds(i * 16, 16)]
                outv[pl.ds(i * 16, 16)] = plsc.load_gather(rowv, [ii])
                return carry

            lax.fori_loop(0, _GP // 16, gather_body, 0)
            pltpu.sync_copy(outv, gexg_hbm.at[wid])

    return k(ppi, gex_idx_padded, gex_x1)


def _bn(x, g, b):
    mu = jnp.mean(x, axis=0, keepdims=True)
    v = jnp.mean((x - mu) ** 2, axis=0, keepdims=True)
    return (x - mu) * lax.rsqrt(v + 1e-5) * g + b


def _prep_drug_body(drug, dose, dur, dW1, db1, dW2, db2, dW3, db3,
                    g1, b1, g3, b3, g4, b4,
                    de_o, dose2_o, dur2_o):
    d0 = _bn(drug[...], g1[...], b1[...])
    h = jnp.maximum(jnp.dot(d0, dW1[...], preferred_element_type=jnp.float32) + db1[...], 0.0)
    h = jnp.maximum(jnp.dot(h, dW2[...], preferred_element_type=jnp.float32) + db2[...], 0.0)
    h = jnp.maximum(jnp.dot(h, dW3[...], preferred_element_type=jnp.float32) + db3[...], 0.0)
    de_o[...] = h
    dose2_o[...] = _bn(dose[...], g3[...], b3[...])
    dur2_o[...] = _bn(dur[...], g4[...], b4[...])


def _prep_gex_body(gexg, adj, g2, b2, gexn_o, logA_o):
    a = adj[...]
    logA_o[...] = jnp.where(a > 0.0, jnp.log(a), -1e30)
    gexn_o[...] = _bn(gexg[...], g2[...], b2[...])


def _gat_layer(x, W, a_s, a_d, gb, logA, ones_col):
    xp = jnp.dot(x, W, preferred_element_type=jnp.float32)
    outs = []
    for h in range(_H):
        xph = xp[:, h * _C:(h + 1) * _C]
        edh = jnp.sum(xph * a_d[:, h * _C:(h + 1) * _C], axis=1, keepdims=True)
        esr = lax.dot_general(a_s[:, h * _C:(h + 1) * _C], xph,
                              (((1,), (1,)), ((), ())),
                              preferred_element_type=jnp.float32)
        e = esr + edh
        em = jnp.maximum(e, 0.2 * e) + logA
        mh = jnp.max(em, axis=1, keepdims=True)
        ex = jnp.exp(em - mh)
        agg = jnp.dot(ex, jnp.concatenate([xph, ones_col], axis=1),
                      preferred_element_type=jnp.float32)
        outs.append(agg[:, :_C] / (agg[:, _C:_C + 1] + 1e-16))
    return jnp.maximum(jnp.concatenate(outs, axis=1) + gb, 0.0)


def _fused_body(gexnT, g2vp, logA_ref, gW0, a0s, a0d, gb0, gW1, a1s, a1d, gb1,
                rW1, rb1, rW2, rb2, rW3, rb3,
                de, dose2, dur2, pW1a, pW1b, pW1c, pb1, pW2, pb2, pW3, pb3,
                out_o):
    b = pl.program_id(0)
    logA = logA_ref[...]
    rowid = lax.broadcasted_iota(jnp.int32, (_B, 1), 0)
    oh = (rowid == b).astype(jnp.float32)
    xcol = jnp.dot(gexnT[...], oh, preferred_element_type=jnp.float32)
    ge = xcol * g2vp[...]

    ones_col = jnp.ones((_GP, 1), jnp.float32)
    h0 = _gat_layer(ge, gW0[...], a0s[...], a0d[...], gb0[...], logA, ones_col)
    h1 = _gat_layer(h0, gW1[...], a1s[...], a1d[...], gb1[...], logA, ones_col)
    hcat = jnp.concatenate([h0, h1], axis=1)

    ro = lax.dot_general(hcat, rW1[...], (((0,), (0,)), ((), ())),
                         preferred_element_type=jnp.float32)
    ro = jnp.maximum(ro + rb1[...], 0.0)
    ro = jnp.maximum(jnp.dot(ro, rW2[...], preferred_element_type=jnp.float32) + rb2[...], 0.0)
    ro = jnp.maximum(jnp.dot(ro, rW3[...], preferred_element_type=jnp.float32) + rb3[...], 0.0)

    t = lax.dot_general(ro, pW1b[...], (((0,), (0,)), ((), ())),
                        preferred_element_type=jnp.float32)
    t = t + jnp.dot(de[0], pW1a[...], preferred_element_type=jnp.float32)
    t = t + dose2[0] * pW1c[0:1, :] + dur2[0] * pW1c[1:2, :]
    t = jnp.maximum(t + pb1[...], 0.0)
    t = jnp.maximum(jnp.dot(t, pW2[...], preferred_element_type=jnp.float32) + pb2[...], 0.0)
    res = jnp.dot(t, pW3[...], preferred_element_type=jnp.float32) + pb3[...]
    out_o[...] = res.reshape(1, 1, res.shape[1])


def kernel(gex_x1, drug_input, dose, duration, get_gex_idxs, ppi_adj, g2v,
           dW1, db1, dW2, db2, dW3, db3, gW0, as0, ad0, gb0, gW1, as1, ad1, gb1,
           rW1, rb1, rW2, rb2, rW3, rb3, pW1, pb1, pW2, pb2, pW3, pb3,
           bn1_g, bn1_b, bn2_g, bn2_b, bn3_g, bn3_b, bn4_g, bn4_b):
    f32 = jnp.float32

    ppi = ppi_adj.astype(jnp.int32)
    idxp = jnp.pad(get_gex_idxs.astype(jnp.int32), (0, _GP - _G))
    adj_flat, gexg = _sc_graph_prep(ppi, idxp, gex_x1.astype(f32))
    adj = adj_flat.reshape(_GP, _GP)

    de, dose2, dur2 = pl.pallas_call(
        _prep_drug_body,
        out_shape=[
            jax.ShapeDtypeStruct((_B, 128), f32),
            jax.ShapeDtypeStruct((_B, 1), f32),
            jax.ShapeDtypeStruct((_B, 1), f32),
        ],
    )(
        drug_input, dose.reshape(_B, 1), duration.reshape(_B, 1),
        dW1, db1.reshape(1, -1), dW2, db2.reshape(1, -1), dW3, db3.reshape(1, -1),
        bn1_g.reshape(1, -1), bn1_b.reshape(1, -1),
        bn3_g.reshape(1, 1), bn3_b.reshape(1, 1),
        bn4_g.reshape(1, 1), bn4_b.reshape(1, 1),
    )

    gexn, logA = pl.pallas_call(
        _prep_gex_body,
        out_shape=[
            jax.ShapeDtypeStruct((_B, _GP), f32),
            jax.ShapeDtypeStruct((_GP, _GP), f32),
        ],
    )(
        gexg, adj,
        jnp.pad(bn2_g, (0, _GP - _G)).reshape(1, _GP),
        jnp.pad(bn2_b, (0, _GP - _G)).reshape(1, _GP),
    )

    gexnT = gexn.T
    g2vp = jnp.pad(g2v, ((0, _GP - _G), (0, 256 - _GVD)))
    gW0p = jnp.pad(gW0, ((0, 256 - _GVD), (0, 0)))
    rW1p = jnp.pad(rW1, ((0, _GP - _G), (0, 0)))
    pW1a = pW1[0:128]
    pW1b = pW1[128:128 + 512]
    pW1c = pW1[128 + 512:]
    r1 = rW1.shape[1]
    r2 = rW2.shape[1]
    p1 = pW1.shape[1]
    p2 = pW2.shape[1]
    nc = pW3.shape[1]

    const2 = lambda blk: pl.BlockSpec(blk, lambda b: (0, 0))
    perb3 = lambda blk: pl.BlockSpec(blk, lambda b: (b, 0, 0))

    out = pl.pallas_call(
        _fused_body,
        grid=(_B,),
        in_specs=[
            const2((_GP, _B)),
            const2((_GP, 256)),
            const2((_GP, _GP)),
            const2((256, _HC)),
            const2((1, _HC)),
            const2((1, _HC)),
            const2((1, _HC)),
            const2((_HC, _HC)),
            const2((1, _HC)),
            const2((1, _HC)),
            const2((1, _HC)),
            const2((_GP, r1)),
            const2((1, r1)),
            const2((r1, r2)),
            const2((1, r2)),
            const2((r2, 1)),
            const2((1, 1)),
            perb3((1, 1, 128)),
            perb3((1, 1, 1)),
            perb3((1, 1, 1)),
            const2((128, p1)),
            const2((512, p1)),
            const2((2, p1)),
            const2((1, p1)),
            const2((p1, p2)),
            const2((1, p2)),
            const2((p2, nc)),
            const2((1, nc)),
        ],
        out_specs=perb3((1, 1, nc)),
        out_shape=jax.ShapeDtypeStruct((_B, 1, nc), f32),
    )(
        gexnT, g2vp, logA, gW0p, as0.reshape(1, _HC), ad0.reshape(1, _HC),
        gb0.reshape(1, _HC), gW1, as1.reshape(1, _HC), ad1.reshape(1, _HC),
        gb1.reshape(1, _HC), rW1p, rb1.reshape(1, r1), rW2, rb2.reshape(1, r2),
        rW3, rb3.reshape(1, 1), de.reshape(_B, 1, 128), dose2.reshape(_B, 1, 1),
        dur2.reshape(_B, 1, 1),
        pW1a, pW1b, pW1c, pb1.reshape(1, p1), pW2, pb2.reshape(1, p2),
        pW3, pb3.reshape(1, nc),
    )
    return out.reshape(_B, nc)
